# baseline (device time: 58207 ns/iter reference)
import jax
import jax.numpy as jnp
from jax import lax
from jax.experimental import pallas as pl
from jax.experimental.pallas import tpu as pltpu

N_DEV = 32
B = 512
D = 256
R = B // N_DEV
N_PHASES = 5


def kernel(x, Win0, Wout0, Win1, Wout1, Win2, Wout2):
    def body(x_ref, win0_ref, wout0_ref, win1_ref, wout1_ref, win2_ref,
             wout2_ref, out_ref, accum_ref, rs_ref, xbuf_ref,
             send_sems, recv_sems):
        me = lax.axis_index("i")

        barrier_sem = pltpu.get_barrier_semaphore()

        def _signal(k, c):
            tgt = lax.rem(me + k, N_DEV)
            pl.semaphore_signal(
                barrier_sem, inc=1,
                device_id=(tgt,), device_id_type=pl.DeviceIdType.MESH,
            )
            return c
        lax.fori_loop(1, N_DEV, _signal, 0)
        pl.semaphore_wait(barrier_sem, N_DEV - 1)

        def rs_send_desc(k, phase):
            tgt = lax.rem(me + k, N_DEV)
            return pltpu.make_async_remote_copy(
                src_ref=accum_ref.at[pl.ds(tgt * R, R), :],
                dst_ref=rs_ref.at[me],
                send_sem=send_sems.at[phase],
                recv_sem=recv_sems.at[phase],
                device_id=(tgt,),
                device_id_type=pl.DeviceIdType.MESH,
            )

        def rs_recv_desc(k, phase):
            src = lax.rem(me + k, N_DEV)
            return pltpu.make_async_remote_copy(
                src_ref=accum_ref.at[pl.ds(0, R), :],
                dst_ref=rs_ref.at[src],
                send_sem=send_sems.at[phase],
                recv_sem=recv_sems.at[phase],
                device_id=(src,),
                device_id_type=pl.DeviceIdType.MESH,
            )

        def ag_send_desc(k, phase):
            tgt = lax.rem(me + k, N_DEV)
            return pltpu.make_async_remote_copy(
                src_ref=xbuf_ref.at[pl.ds(me * R, R), :],
                dst_ref=xbuf_ref.at[pl.ds(me * R, R), :],
                send_sem=send_sems.at[phase],
                recv_sem=recv_sems.at[phase],
                device_id=(tgt,),
                device_id_type=pl.DeviceIdType.MESH,
            )

        def ag_recv_desc(k, phase):
            src = lax.rem(me + k, N_DEV)
            return pltpu.make_async_remote_copy(
                src_ref=xbuf_ref.at[pl.ds(0, R), :],
                dst_ref=xbuf_ref.at[pl.ds(src * R, R), :],
                send_sem=send_sems.at[phase],
                recv_sem=recv_sems.at[phase],
                device_id=(src,),
                device_id_type=pl.DeviceIdType.MESH,
            )

        def rs_phase(phase):
            def _start(k, c):
                rs_send_desc(k, phase).start()
                return c
            lax.fori_loop(1, N_DEV, _start, 0)
            rs_ref[me] = accum_ref[pl.ds(me * R, R), :]

            def _wait(k, c):
                rs_send_desc(k, phase).wait_send()
                rs_recv_desc(k, phase).wait_recv()
                return c
            lax.fori_loop(1, N_DEV, _wait, 0)
            return jnp.sum(rs_ref[...], axis=0)

        def ag_phase(phase, y):
            xbuf_ref[pl.ds(me * R, R), :] = y

            def _start(k, c):
                ag_send_desc(k, phase).start()
                return c
            lax.fori_loop(1, N_DEV, _start, 0)

            def _wait(k, c):
                ag_send_desc(k, phase).wait_send()
                ag_recv_desc(k, phase).wait_recv()
                return c
            lax.fori_loop(1, N_DEV, _wait, 0)

        def layer(xv, win_ref, wout_ref):
            h = jnp.dot(xv.astype(jnp.bfloat16),
                        win_ref[...].astype(jnp.bfloat16),
                        preferred_element_type=jnp.float32)
            h = jnp.maximum(h, 0.0)
            return jnp.dot(h.astype(jnp.bfloat16),
                           wout_ref[...].astype(jnp.bfloat16),
                           preferred_element_type=jnp.float32)

        accum_ref[...] = layer(x_ref[...], win0_ref, wout0_ref)
        y0 = rs_phase(0)
        ag_phase(1, y0)

        accum_ref[...] = layer(xbuf_ref[...], win1_ref, wout1_ref)
        y1 = rs_phase(2)
        ag_phase(3, y1)

        accum_ref[...] = layer(xbuf_ref[...], win2_ref, wout2_ref)
        out_ref[...] = rs_phase(4)

    return pl.pallas_call(
        body,
        out_shape=jax.ShapeDtypeStruct((R, D), jnp.float32),
        in_specs=[pl.BlockSpec(memory_space=pltpu.VMEM)] * 7,
        out_specs=pl.BlockSpec(memory_space=pltpu.VMEM),
        scratch_shapes=[
            pltpu.VMEM((B, D), jnp.float32),
            pltpu.VMEM((N_DEV, R, D), jnp.float32),
            pltpu.VMEM((B, D), jnp.float32),
            pltpu.SemaphoreType.DMA((N_PHASES,)),
            pltpu.SemaphoreType.DMA((N_PHASES,)),
        ],
        compiler_params=pltpu.CompilerParams(collective_id=0),
    )(x, Win0, Wout0, Win1, Wout1, Win2, Wout2)


# device time: 46111 ns/iter; 1.2623x vs baseline; 1.2623x over previous
import jax
import jax.numpy as jnp
from jax import lax
from jax.experimental import pallas as pl
from jax.experimental.pallas import tpu as pltpu

N_DEV = 32
B = 512
D = 256
R = B // N_DEV
N_PHASES = 5


def kernel(x, Win0, Wout0, Win1, Wout1, Win2, Wout2):
    def body(x_ref, win0_ref, wout0_ref, win1_ref, wout1_ref, win2_ref,
             wout2_ref, out_ref, accum_ref, rs_ref, xbuf_ref,
             send_sems, recv_sems):
        me = lax.axis_index("i")

        barrier_sem = pltpu.get_barrier_semaphore()

        def _signal(k, c):
            tgt = lax.rem(me + k, N_DEV)
            pl.semaphore_signal(
                barrier_sem, inc=1,
                device_id=(tgt,), device_id_type=pl.DeviceIdType.MESH,
            )
            return c
        lax.fori_loop(1, N_DEV, _signal, 0)
        pl.semaphore_wait(barrier_sem, N_DEV - 1)

        def rs_send_desc(k, phase):
            tgt = lax.rem(me + k, N_DEV)
            return pltpu.make_async_remote_copy(
                src_ref=accum_ref.at[pl.ds(tgt * R, R), :],
                dst_ref=rs_ref.at[me],
                send_sem=send_sems.at[phase],
                recv_sem=recv_sems.at[phase],
                device_id=(tgt,),
                device_id_type=pl.DeviceIdType.MESH,
            )

        def rs_recv_desc(k, phase):
            src = lax.rem(me + k, N_DEV)
            return pltpu.make_async_remote_copy(
                src_ref=accum_ref.at[pl.ds(0, R), :],
                dst_ref=rs_ref.at[src],
                send_sem=send_sems.at[phase],
                recv_sem=recv_sems.at[phase],
                device_id=(src,),
                device_id_type=pl.DeviceIdType.MESH,
            )

        def ag_send_desc(k, phase):
            tgt = lax.rem(me + k, N_DEV)
            return pltpu.make_async_remote_copy(
                src_ref=xbuf_ref.at[pl.ds(me * R, R), :],
                dst_ref=xbuf_ref.at[pl.ds(me * R, R), :],
                send_sem=send_sems.at[phase],
                recv_sem=recv_sems.at[phase],
                device_id=(tgt,),
                device_id_type=pl.DeviceIdType.MESH,
            )

        def ag_recv_desc(k, phase):
            src = lax.rem(me + k, N_DEV)
            return pltpu.make_async_remote_copy(
                src_ref=xbuf_ref.at[pl.ds(0, R), :],
                dst_ref=xbuf_ref.at[pl.ds(src * R, R), :],
                send_sem=send_sems.at[phase],
                recv_sem=recv_sems.at[phase],
                device_id=(src,),
                device_id_type=pl.DeviceIdType.MESH,
            )

        def rs_phase(phase):
            def _start(k, c):
                rs_send_desc(k, phase).start()
                return c
            lax.fori_loop(1, N_DEV, _start, 0)
            rs_ref[me] = accum_ref[pl.ds(me * R, R), :]

            def _wait(k, c):
                rs_send_desc(k, phase).wait_send()
                rs_recv_desc(k, phase).wait_recv()
                return c
            lax.fori_loop(1, N_DEV, _wait, 0)
            return jnp.sum(rs_ref[...].astype(jnp.float32), axis=0)

        def ag_phase(phase, y):
            xbuf_ref[pl.ds(me * R, R), :] = y.astype(jnp.bfloat16)

            def _start(k, c):
                ag_send_desc(k, phase).start()
                return c
            lax.fori_loop(1, N_DEV, _start, 0)

            def _wait(k, c):
                ag_send_desc(k, phase).wait_send()
                ag_recv_desc(k, phase).wait_recv()
                return c
            lax.fori_loop(1, N_DEV, _wait, 0)

        def layer(xv, win_ref, wout_ref):
            h = jnp.dot(xv.astype(jnp.bfloat16),
                        win_ref[...].astype(jnp.bfloat16),
                        preferred_element_type=jnp.float32)
            h = jnp.maximum(h, 0.0)
            p = jnp.dot(h.astype(jnp.bfloat16),
                        wout_ref[...].astype(jnp.bfloat16),
                        preferred_element_type=jnp.float32)
            return p.astype(jnp.bfloat16)

        accum_ref[...] = layer(x_ref[...], win0_ref, wout0_ref)
        y0 = rs_phase(0)
        ag_phase(1, y0)

        accum_ref[...] = layer(xbuf_ref[...], win1_ref, wout1_ref)
        y1 = rs_phase(2)
        ag_phase(3, y1)

        accum_ref[...] = layer(xbuf_ref[...], win2_ref, wout2_ref)
        out_ref[...] = rs_phase(4)

    return pl.pallas_call(
        body,
        out_shape=jax.ShapeDtypeStruct((R, D), jnp.float32),
        in_specs=[pl.BlockSpec(memory_space=pltpu.VMEM)] * 7,
        out_specs=pl.BlockSpec(memory_space=pltpu.VMEM),
        scratch_shapes=[
            pltpu.VMEM((B, D), jnp.bfloat16),
            pltpu.VMEM((N_DEV, R, D), jnp.bfloat16),
            pltpu.VMEM((B, D), jnp.bfloat16),
            pltpu.SemaphoreType.DMA((N_PHASES,)),
            pltpu.SemaphoreType.DMA((N_PHASES,)),
        ],
        compiler_params=pltpu.CompilerParams(collective_id=0),
    )(x, Win0, Wout0, Win1, Wout1, Win2, Wout2)


# device time: 46099 ns/iter; 1.2627x vs baseline; 1.0003x over previous
import jax
import jax.numpy as jnp
from jax import lax
from jax.experimental import pallas as pl
from jax.experimental.pallas import tpu as pltpu

N_DEV = 32
B = 512
D = 256
R = B // N_DEV
N_PHASES = 5


def kernel(x, Win0, Wout0, Win1, Wout1, Win2, Wout2):
    def body(x_ref, win0_ref, wout0_ref, win1_ref, wout1_ref, win2_ref,
             wout2_ref, out_ref, accum_ref, rs_ref, xbuf_ref,
             send_sems, recv_sems):
        me = lax.axis_index("i")

        barrier_sem = pltpu.get_barrier_semaphore()

        def _signal(k, c):
            tgt = lax.rem(me + k, N_DEV)
            pl.semaphore_signal(
                barrier_sem, inc=1,
                device_id=(tgt,), device_id_type=pl.DeviceIdType.MESH,
            )
            return c
        lax.fori_loop(1, N_DEV, _signal, 0)
        pl.semaphore_wait(barrier_sem, N_DEV - 1)

        def rs_send_desc(k, phase):
            tgt = lax.rem(me + k, N_DEV)
            return pltpu.make_async_remote_copy(
                src_ref=accum_ref.at[pl.ds(tgt * R, R), :],
                dst_ref=rs_ref.at[me],
                send_sem=send_sems.at[phase],
                recv_sem=recv_sems.at[phase],
                device_id=(tgt,),
                device_id_type=pl.DeviceIdType.MESH,
            )

        def rs_recv_desc(k, phase):
            src = lax.rem(me + k, N_DEV)
            return pltpu.make_async_remote_copy(
                src_ref=accum_ref.at[pl.ds(0, R), :],
                dst_ref=rs_ref.at[src],
                send_sem=send_sems.at[phase],
                recv_sem=recv_sems.at[phase],
                device_id=(src,),
                device_id_type=pl.DeviceIdType.MESH,
            )

        def ag_send_desc(k, phase):
            tgt = lax.rem(me + k, N_DEV)
            return pltpu.make_async_remote_copy(
                src_ref=xbuf_ref.at[pl.ds(me * R, R), :],
                dst_ref=xbuf_ref.at[pl.ds(me * R, R), :],
                send_sem=send_sems.at[phase],
                recv_sem=recv_sems.at[phase],
                device_id=(tgt,),
                device_id_type=pl.DeviceIdType.MESH,
            )

        def ag_recv_desc(k, phase):
            src = lax.rem(me + k, N_DEV)
            return pltpu.make_async_remote_copy(
                src_ref=xbuf_ref.at[pl.ds(0, R), :],
                dst_ref=xbuf_ref.at[pl.ds(src * R, R), :],
                send_sem=send_sems.at[phase],
                recv_sem=recv_sems.at[phase],
                device_id=(src,),
                device_id_type=pl.DeviceIdType.MESH,
            )

        def rs_phase(phase):
            for k in range(1, N_DEV):
                rs_send_desc(k, phase).start()
            rs_ref[me] = accum_ref[pl.ds(me * R, R), :]
            for k in range(1, N_DEV):
                rs_send_desc(k, phase).wait_send()
                rs_recv_desc(k, phase).wait_recv()
            return jnp.sum(rs_ref[...].astype(jnp.float32), axis=0)

        def ag_phase(phase, y):
            xbuf_ref[pl.ds(me * R, R), :] = y.astype(jnp.bfloat16)
            for k in range(1, N_DEV):
                ag_send_desc(k, phase).start()
            for k in range(1, N_DEV):
                ag_send_desc(k, phase).wait_send()
                ag_recv_desc(k, phase).wait_recv()

        def layer(xv, win_ref, wout_ref):
            h = jnp.dot(xv.astype(jnp.bfloat16),
                        win_ref[...].astype(jnp.bfloat16),
                        preferred_element_type=jnp.float32)
            h = jnp.maximum(h, 0.0)
            p = jnp.dot(h.astype(jnp.bfloat16),
                        wout_ref[...].astype(jnp.bfloat16),
                        preferred_element_type=jnp.float32)
            return p.astype(jnp.bfloat16)

        accum_ref[...] = layer(x_ref[...], win0_ref, wout0_ref)
        y0 = rs_phase(0)
        ag_phase(1, y0)

        accum_ref[...] = layer(xbuf_ref[...], win1_ref, wout1_ref)
        y1 = rs_phase(2)
        ag_phase(3, y1)

        accum_ref[...] = layer(xbuf_ref[...], win2_ref, wout2_ref)
        out_ref[...] = rs_phase(4)

    return pl.pallas_call(
        body,
        out_shape=jax.ShapeDtypeStruct((R, D), jnp.float32),
        in_specs=[pl.BlockSpec(memory_space=pltpu.VMEM)] * 7,
        out_specs=pl.BlockSpec(memory_space=pltpu.VMEM),
        scratch_shapes=[
            pltpu.VMEM((B, D), jnp.bfloat16),
            pltpu.VMEM((N_DEV, R, D), jnp.bfloat16),
            pltpu.VMEM((B, D), jnp.bfloat16),
            pltpu.SemaphoreType.DMA((N_PHASES,)),
            pltpu.SemaphoreType.DMA((N_PHASES,)),
        ],
        compiler_params=pltpu.CompilerParams(collective_id=0),
    )(x, Win0, Wout0, Win1, Wout1, Win2, Wout2)
